# Initial kernel scaffold; baseline (speedup 1.0000x reference)
import jax
import jax.numpy as jnp
from jax.experimental import pallas as pl


def kernel(d, d_star):
    raise NotImplementedError("write your pallas kernel here")


# speedup vs baseline: 23.2421x; 23.2421x over previous
import functools

import jax
import jax.numpy as jnp
from jax import lax
from jax.experimental import pallas as pl
from jax.experimental.pallas import tpu as pltpu
from jax.experimental.pallas import tpu_sc as plsc

B = 8
M = 512 * 512
M80 = int(0.8 * M)
NB = 8192
NSEG = 4
SEG = M // NSEG
CHUNK = 8192
NCHUNK = SEG // CHUNK
VPC = CHUNK // 16


def _sc_mesh():
    return plsc.VectorSubcoreMesh(core_axis_name="c", subcore_axis_name="s")


def _zero_vmem(ref, n):
    zero = jnp.zeros((16,), jnp.float32)

    def body(i, _):
        ref[pl.ds(i * 16, 16)] = zero
        return 0

    lax.fori_loop(0, n // 16, body, 0, unroll=8)


def _hist_add(hist, b):
    plsc.addupdate_scatter(hist, [b], jnp.ones((16,), jnp.float32))


def _value_hist_kernel(d_hbm, ds_hbm, out_hbm, hist_d, hist_ds, buf):
    c = lax.axis_index("c")
    s = lax.axis_index("s")
    row = c * (16 // NSEG) + s // NSEG
    seg = s % NSEG
    base = seg * SEG

    _zero_vmem(hist_d, NB)
    _zero_vmem(hist_ds, NB)

    for src, hist in ((d_hbm, hist_d), (ds_hbm, hist_ds)):
        for ch in range(NCHUNK):
            pltpu.sync_copy(src.at[row, pl.ds(base + ch * CHUNK, CHUNK)], buf)

            def body(j, _):
                x = buf[pl.ds(j * 16, 16)]
                b = jnp.minimum((x * float(NB)).astype(jnp.int32), NB - 1)
                _hist_add(hist, b)
                return 0

            lax.fori_loop(0, VPC, body, 0, unroll=4)

    pltpu.sync_copy(hist_d, out_hbm.at[row, seg, 0])
    pltpu.sync_copy(hist_ds, out_hbm.at[row, seg, 1])


def _diff_hist_kernel(d_hbm, ds_hbm, p_hbm, out_hbm, hist, buf_d, buf_ds, pbuf):
    c = lax.axis_index("c")
    s = lax.axis_index("s")
    row = c * (16 // NSEG) + s // NSEG
    seg = s % NSEG
    base = seg * SEG

    _zero_vmem(hist, NB)
    pltpu.sync_copy(p_hbm.at[row], pbuf)

    p = pbuf[pl.ds(0, 16)]
    lanes = lax.iota(jnp.int32, 16)

    def col(i):
        return jnp.sum(jnp.where(lanes == i, p, 0.0), axis=0)

    t_d = col(0)
    t_ds = col(1)
    inv_sd = col(2)
    inv_sds = col(3)
    scale = col(4)

    for ch in range(NCHUNK):
        pltpu.sync_copy(d_hbm.at[row, pl.ds(base + ch * CHUNK, CHUNK)], buf_d)
        pltpu.sync_copy(ds_hbm.at[row, pl.ds(base + ch * CHUNK, CHUNK)], buf_ds)

        def body(j, _):
            xd = buf_d[pl.ds(j * 16, 16)]
            xs = buf_ds[pl.ds(j * 16, 16)]
            diff = jnp.abs((xd - t_d) * inv_sd - (xs - t_ds) * inv_sds)
            b = jnp.minimum((diff * scale).astype(jnp.int32), NB - 1)
            _hist_add(hist, b)
            return 0

        lax.fori_loop(0, VPC, body, 0, unroll=4)

    pltpu.sync_copy(hist, out_hbm.at[row, seg])


def _make_sc_kernels():
    mesh = _sc_mesh()
    cp = pltpu.CompilerParams(needs_layout_passes=False)
    k1 = functools.partial(
        pl.kernel,
        out_type=jax.ShapeDtypeStruct((B, NSEG, 2, NB), jnp.float32),
        mesh=mesh,
        compiler_params=cp,
        scratch_types=[
            pltpu.VMEM((NB,), jnp.float32),
            pltpu.VMEM((NB,), jnp.float32),
            pltpu.VMEM((CHUNK,), jnp.float32),
        ],
    )(_value_hist_kernel)
    k3 = functools.partial(
        pl.kernel,
        out_type=jax.ShapeDtypeStruct((B, NSEG, NB), jnp.float32),
        mesh=mesh,
        compiler_params=cp,
        scratch_types=[
            pltpu.VMEM((NB,), jnp.float32),
            pltpu.VMEM((CHUNK,), jnp.float32),
            pltpu.VMEM((CHUNK,), jnp.float32),
            pltpu.VMEM((128,), jnp.float32),
        ],
    )(_diff_hist_kernel)
    return k1, k3


def _cum_hier(cnt8):
    nchunks = NB // 128
    nrows = B * nchunks
    cnt2 = cnt8.reshape(nrows, 128)
    i128 = lax.broadcasted_iota(jnp.int32, (128, 128), 0)
    j128 = lax.broadcasted_iota(jnp.int32, (128, 128), 1)
    tri = (i128 <= j128).astype(jnp.float32)
    lanecum = jax.lax.dot(cnt2, tri, precision=jax.lax.Precision.HIGHEST)
    ii = lax.broadcasted_iota(jnp.int32, (nrows, nrows), 0)
    jj = lax.broadcasted_iota(jnp.int32, (nrows, nrows), 1)
    prevchunk = ((ii // nchunks == jj // nchunks) & (jj < ii)).astype(jnp.float32)
    a = jax.lax.dot(prevchunk, cnt2, precision=jax.lax.Precision.HIGHEST)
    ones = jnp.ones((128, 128), jnp.float32)
    cum2 = lanecum + jax.lax.dot(a, ones, precision=jax.lax.Precision.HIGHEST)
    return cum2.reshape(B, NB)


def _rank_value(cnt, cum, bidx, k):
    below = (cum < (k + 0.5)).astype(jnp.float32)
    bstar = jnp.sum(below, axis=1, keepdims=True)
    nbelow = jnp.sum(cnt * below, axis=1, keepdims=True)
    onehot = (bidx == bstar).astype(jnp.float32)
    cntb = jnp.sum(cnt * onehot, axis=1, keepdims=True)
    cntb = jnp.maximum(cntb, 1.0)
    return bstar + (k - nbelow + 0.5) / cntb


def _params_kernel(h_ref, out_ref):
    h = h_ref[...]
    cnt_d = jnp.sum(h[:, :, 0, :], axis=1)
    cnt_ds = jnp.sum(h[:, :, 1, :], axis=1)

    bidx = lax.broadcasted_iota(jnp.int32, (B, NB), 1).astype(jnp.float32)
    width = 1.0 / NB
    centers = (bidx + 0.5) * width

    def stats(cnt):
        cum = _cum_hier(cnt)
        v0 = _rank_value(cnt, cum, bidx, float(M // 2 - 1))
        v1 = _rank_value(cnt, cum, bidx, float(M // 2))
        t = (v0 + v1) * (0.5 * width)
        absdev = jnp.sum(cnt * jnp.abs(centers - t), axis=1, keepdims=True)
        s = (jnp.sum(absdev) + 1e-08) / M
        return t, s

    t_d, s_d = stats(cnt_d)
    t_ds, s_ds = stats(cnt_ds)

    inv_sd = jnp.full((B, 1), 1.0 / s_d, jnp.float32)
    inv_sds = jnp.full((B, 1), 1.0 / s_ds, jnp.float32)
    r_bound = (jnp.maximum(t_d, 1.0 - t_d) / s_d
               + jnp.maximum(t_ds, 1.0 - t_ds) / s_ds) * (1.0 + 2.0 ** -10)
    scale = NB / r_bound
    widthd = r_bound / NB
    pad = jnp.zeros((B, 128 - 6), jnp.float32)
    out_ref[...] = jnp.concatenate(
        [t_d, t_ds, inv_sd, inv_sds, scale, widthd, pad], axis=1)


def _loss_kernel(h_ref, p_ref, out_ref):
    h = h_ref[...]
    cnt = jnp.sum(h, axis=1)
    widthd = p_ref[:, 5:6]

    cum = _cum_hier(cnt)
    bidx = lax.broadcasted_iota(jnp.int32, (B, NB), 1).astype(jnp.float32)
    below = (cum < (M80 - 0.5)).astype(jnp.float32)
    bstar = jnp.sum(below, axis=1, keepdims=True)
    nbelow = jnp.sum(cnt * below, axis=1, keepdims=True)
    onehot = (bidx == bstar).astype(jnp.float32)
    cntb = jnp.maximum(jnp.sum(cnt * onehot, axis=1, keepdims=True), 1.0)

    centers = (bidx + 0.5) * widthd
    s_below = jnp.sum(cnt * centers * below, axis=1, keepdims=True)
    m = M80 - nbelow
    left = bstar * widthd
    s_in = m * left + widthd * m * m / (2.0 * cntb)
    loss = jnp.sum(s_below + s_in) / (2.0 * M)
    out_ref[...] = loss.reshape(1, 1)


def kernel(d, d_star):
    d2 = d.reshape(B, M)
    ds2 = d_star.reshape(B, M)
    k1, k3 = _make_sc_kernels()

    h1 = k1(d2, ds2)
    params = pl.pallas_call(
        _params_kernel,
        out_shape=jax.ShapeDtypeStruct((B, 128), jnp.float32),
    )(h1)
    h2 = k3(d2, ds2, params)
    out = pl.pallas_call(
        _loss_kernel,
        out_shape=jax.ShapeDtypeStruct((1, 1), jnp.float32),
    )(h2, params)
    return out.reshape(())


# speedup vs baseline: 27.4982x; 1.1831x over previous
import functools

import jax
import jax.numpy as jnp
from jax import lax
from jax.experimental import pallas as pl
from jax.experimental.pallas import tpu as pltpu
from jax.experimental.pallas import tpu_sc as plsc

B = 8
M = 512 * 512
M80 = int(0.8 * M)
NB = 8192
NSEG = 4
ROWS_PER_SEG = 512 // NSEG
CROWS = 16
NCHUNK = ROWS_PER_SEG // CROWS
VPC = CROWS * 512 // 16
VPR = 512 // 16


def _sc_mesh():
    return plsc.VectorSubcoreMesh(core_axis_name="c", subcore_axis_name="s")


def _zero_vmem(ref, n):
    zero = jnp.zeros((16,), jnp.float32)

    def body(i, _):
        ref[pl.ds(i * 16, 16)] = zero
        return 0

    lax.fori_loop(0, n // 16, body, 0, unroll=8)


def _hist_add(hist, b):
    plsc.addupdate_scatter(hist, [b], jnp.ones((16,), jnp.float32))


def _value_hist_kernel(d_hbm, ds_hbm, out_hbm, hist_d, hist_ds, buf0, buf1,
                       sem0, sem1):
    c = lax.axis_index("c")
    s = lax.axis_index("s")
    row = c * (16 // NSEG) + s // NSEG
    seg = s % NSEG
    base = seg * ROWS_PER_SEG

    _zero_vmem(hist_d, NB)
    _zero_vmem(hist_ds, NB)

    bufs = (buf0, buf1)
    sems = (sem0, sem1)
    plan = [(d_hbm, hist_d, ch) for ch in range(NCHUNK)]
    plan += [(ds_hbm, hist_ds, ch) for ch in range(NCHUNK)]

    def start(i):
        src, _, ch = plan[i]
        return pltpu.async_copy(
            src.at[row, pl.ds(base + ch * CROWS, CROWS), :], bufs[i % 2],
            sems[i % 2])

    cp = start(0)
    for i in range(len(plan)):
        cp.wait()
        if i + 1 < len(plan):
            cp = start(i + 1)
        buf = bufs[i % 2]
        hist = plan[i][1]

        def body(j, _):
            x = buf[j // VPR, pl.ds((j % VPR) * 16, 16)]
            b = jnp.minimum((x * float(NB)).astype(jnp.int32), NB - 1)
            _hist_add(hist, b)
            return 0

        lax.fori_loop(0, VPC, body, 0, unroll=4)

    pltpu.sync_copy(hist_d, out_hbm.at[row, seg, 0])
    pltpu.sync_copy(hist_ds, out_hbm.at[row, seg, 1])


def _diff_hist_kernel(d_hbm, ds_hbm, p_hbm, out_hbm, hist, buf_d0, buf_d1,
                      buf_ds0, buf_ds1, pbuf, sem_d0, sem_d1, sem_ds0,
                      sem_ds1):
    c = lax.axis_index("c")
    s = lax.axis_index("s")
    row = c * (16 // NSEG) + s // NSEG
    seg = s % NSEG
    base = seg * ROWS_PER_SEG

    _zero_vmem(hist, NB)
    pltpu.sync_copy(p_hbm.at[row], pbuf)

    p = pbuf[pl.ds(0, 16)]
    lanes = lax.iota(jnp.int32, 16)

    def col(i):
        return jnp.sum(jnp.where(lanes == i, p, 0.0), axis=0)

    t_d = col(0)
    t_ds = col(1)
    inv_sd = col(2)
    inv_sds = col(3)
    scale = col(4)

    bufs_d = (buf_d0, buf_d1)
    bufs_ds = (buf_ds0, buf_ds1)
    sems_d = (sem_d0, sem_d1)
    sems_ds = (sem_ds0, sem_ds1)

    def start(ch):
        sl = pl.ds(base + ch * CROWS, CROWS)
        return (pltpu.async_copy(d_hbm.at[row, sl, :], bufs_d[ch % 2],
                                 sems_d[ch % 2]),
                pltpu.async_copy(ds_hbm.at[row, sl, :], bufs_ds[ch % 2],
                                 sems_ds[ch % 2]))

    cps = start(0)
    for ch in range(NCHUNK):
        cps[0].wait()
        cps[1].wait()
        if ch + 1 < NCHUNK:
            cps = start(ch + 1)
        bd = bufs_d[ch % 2]
        bs = bufs_ds[ch % 2]

        def body(j, _):
            r = j // VPR
            col2 = (j % VPR) * 16
            xd = bd[r, pl.ds(col2, 16)]
            xs = bs[r, pl.ds(col2, 16)]
            diff = jnp.abs((xd - t_d) * inv_sd - (xs - t_ds) * inv_sds)
            b = jnp.minimum((diff * scale).astype(jnp.int32), NB - 1)
            _hist_add(hist, b)
            return 0

        lax.fori_loop(0, VPC, body, 0, unroll=4)

    pltpu.sync_copy(hist, out_hbm.at[row, seg])


def _make_sc_kernels():
    mesh = _sc_mesh()
    cp = pltpu.CompilerParams(needs_layout_passes=False,
                              use_tc_tiling_on_sc=True)
    k1 = functools.partial(
        pl.kernel,
        out_type=jax.ShapeDtypeStruct((B, NSEG, 2, NB), jnp.float32),
        mesh=mesh,
        compiler_params=cp,
        scratch_types=[
            pltpu.VMEM((NB,), jnp.float32),
            pltpu.VMEM((NB,), jnp.float32),
            pltpu.VMEM((CROWS, 512), jnp.float32),
            pltpu.VMEM((CROWS, 512), jnp.float32),
            pltpu.SemaphoreType.DMA,
            pltpu.SemaphoreType.DMA,
        ],
    )(_value_hist_kernel)
    k3 = functools.partial(
        pl.kernel,
        out_type=jax.ShapeDtypeStruct((B, NSEG, NB), jnp.float32),
        mesh=mesh,
        compiler_params=cp,
        scratch_types=[
            pltpu.VMEM((NB,), jnp.float32),
            pltpu.VMEM((CROWS, 512), jnp.float32),
            pltpu.VMEM((CROWS, 512), jnp.float32),
            pltpu.VMEM((CROWS, 512), jnp.float32),
            pltpu.VMEM((CROWS, 512), jnp.float32),
            pltpu.VMEM((128,), jnp.float32),
            pltpu.SemaphoreType.DMA,
            pltpu.SemaphoreType.DMA,
            pltpu.SemaphoreType.DMA,
            pltpu.SemaphoreType.DMA,
        ],
    )(_diff_hist_kernel)
    return k1, k3


def _cum_hier(cnt8):
    nchunks = NB // 128
    nrows = B * nchunks
    cnt2 = cnt8.reshape(nrows, 128)
    i128 = lax.broadcasted_iota(jnp.int32, (128, 128), 0)
    j128 = lax.broadcasted_iota(jnp.int32, (128, 128), 1)
    tri = (i128 <= j128).astype(jnp.float32)
    lanecum = jax.lax.dot(cnt2, tri, precision=jax.lax.Precision.HIGHEST)
    ii = lax.broadcasted_iota(jnp.int32, (nrows, nrows), 0)
    jj = lax.broadcasted_iota(jnp.int32, (nrows, nrows), 1)
    prevchunk = ((ii // nchunks == jj // nchunks) & (jj < ii)).astype(jnp.float32)
    a = jax.lax.dot(prevchunk, cnt2, precision=jax.lax.Precision.HIGHEST)
    ones = jnp.ones((128, 128), jnp.float32)
    cum2 = lanecum + jax.lax.dot(a, ones, precision=jax.lax.Precision.HIGHEST)
    return cum2.reshape(B, NB)


def _rank_value(cnt, cum, bidx, k):
    below = (cum < (k + 0.5)).astype(jnp.float32)
    bstar = jnp.sum(below, axis=1, keepdims=True)
    nbelow = jnp.sum(cnt * below, axis=1, keepdims=True)
    onehot = (bidx == bstar).astype(jnp.float32)
    cntb = jnp.sum(cnt * onehot, axis=1, keepdims=True)
    cntb = jnp.maximum(cntb, 1.0)
    return bstar + (k - nbelow + 0.5) / cntb


def _params_kernel(h_ref, out_ref):
    h = h_ref[...]
    cnt_d = jnp.sum(h[:, :, 0, :], axis=1)
    cnt_ds = jnp.sum(h[:, :, 1, :], axis=1)

    bidx = lax.broadcasted_iota(jnp.int32, (B, NB), 1).astype(jnp.float32)
    width = 1.0 / NB
    centers = (bidx + 0.5) * width

    def stats(cnt):
        cum = _cum_hier(cnt)
        v0 = _rank_value(cnt, cum, bidx, float(M // 2 - 1))
        v1 = _rank_value(cnt, cum, bidx, float(M // 2))
        t = (v0 + v1) * (0.5 * width)
        absdev = jnp.sum(cnt * jnp.abs(centers - t), axis=1, keepdims=True)
        s = (jnp.sum(absdev) + 1e-08) / M
        return t, s

    t_d, s_d = stats(cnt_d)
    t_ds, s_ds = stats(cnt_ds)

    inv_sd = jnp.full((B, 1), 1.0 / s_d, jnp.float32)
    inv_sds = jnp.full((B, 1), 1.0 / s_ds, jnp.float32)
    r_bound = (jnp.maximum(t_d, 1.0 - t_d) / s_d
               + jnp.maximum(t_ds, 1.0 - t_ds) / s_ds) * (1.0 + 2.0 ** -10)
    scale = NB / r_bound
    widthd = r_bound / NB
    pad = jnp.zeros((B, 128 - 6), jnp.float32)
    out_ref[...] = jnp.concatenate(
        [t_d, t_ds, inv_sd, inv_sds, scale, widthd, pad], axis=1)


def _loss_kernel(h_ref, p_ref, out_ref):
    h = h_ref[...]
    cnt = jnp.sum(h, axis=1)
    widthd = p_ref[:, 5:6]

    cum = _cum_hier(cnt)
    bidx = lax.broadcasted_iota(jnp.int32, (B, NB), 1).astype(jnp.float32)
    below = (cum < (M80 - 0.5)).astype(jnp.float32)
    bstar = jnp.sum(below, axis=1, keepdims=True)
    nbelow = jnp.sum(cnt * below, axis=1, keepdims=True)
    onehot = (bidx == bstar).astype(jnp.float32)
    cntb = jnp.maximum(jnp.sum(cnt * onehot, axis=1, keepdims=True), 1.0)

    centers = (bidx + 0.5) * widthd
    s_below = jnp.sum(cnt * centers * below, axis=1, keepdims=True)
    m = M80 - nbelow
    left = bstar * widthd
    s_in = m * left + widthd * m * m / (2.0 * cntb)
    loss = jnp.sum(s_below + s_in) / (2.0 * M)
    out_ref[...] = loss.reshape(1, 1)


def kernel(d, d_star):
    d2 = d.reshape(B, 512, 512)
    ds2 = d_star.reshape(B, 512, 512)
    k1, k3 = _make_sc_kernels()

    h1 = k1(d2, ds2)
    params = pl.pallas_call(
        _params_kernel,
        out_shape=jax.ShapeDtypeStruct((B, 128), jnp.float32),
    )(h1)
    h2 = k3(d2, ds2, params)
    out = pl.pallas_call(
        _loss_kernel,
        out_shape=jax.ShapeDtypeStruct((1, 1), jnp.float32),
    )(h2, params)
    return out.reshape(())


# speedup vs baseline: 76.9050x; 2.7967x over previous
import functools

import jax
import jax.numpy as jnp
from jax import lax
from jax.experimental import pallas as pl
from jax.experimental.pallas import tpu as pltpu
from jax.experimental.pallas import tpu_sc as plsc

B = 8
M = 512 * 512
M80 = int(0.8 * M)
NB = 8192
NSEG = 4
ROWS_PER_SEG = 512 // NSEG
CROWS = 16
NCHUNK = ROWS_PER_SEG // CROWS
VPC = CROWS * 512 // 16
VPR = 512 // 16


def _sc_mesh():
    return plsc.VectorSubcoreMesh(core_axis_name="c", subcore_axis_name="s")


def _zero_vmem(ref, n):
    zero = jnp.zeros((16,), jnp.float32)

    def body(i, _):
        ref[pl.ds(i * 16, 16)] = zero
        return 0

    lax.fori_loop(0, n // 16, body, 0, unroll=8)


def _hist_add(hist, b):
    plsc.addupdate_scatter(hist, [b], jnp.ones((16,), jnp.float32))


def _value_hist_kernel(d_hbm, ds_hbm, out_hbm, hist_d, hist_ds, buf0, buf1,
                       sem0, sem1):
    c = lax.axis_index("c")
    s = lax.axis_index("s")
    row = c * (16 // NSEG) + s // NSEG
    seg = s % NSEG
    base = seg * ROWS_PER_SEG

    _zero_vmem(hist_d, NB)
    _zero_vmem(hist_ds, NB)

    bufs = (buf0, buf1)
    sems = (sem0, sem1)
    plan = [(d_hbm, hist_d, ch) for ch in range(NCHUNK)]
    plan += [(ds_hbm, hist_ds, ch) for ch in range(NCHUNK)]

    def start(i):
        src, _, ch = plan[i]
        return pltpu.async_copy(
            src.at[row, pl.ds(base + ch * CROWS, CROWS), :], bufs[i % 2],
            sems[i % 2])

    cp = start(0)
    for i in range(len(plan)):
        cp.wait()
        if i + 1 < len(plan):
            cp = start(i + 1)
        buf = bufs[i % 2]
        hist = plan[i][1]

        @plsc.parallel_loop(0, VPC, 1, unroll=8)
        def body(j):
            x = buf[j // VPR, pl.ds((j % VPR) * 16, 16)]
            b = jnp.minimum((x * float(NB)).astype(jnp.int32), NB - 1)
            _hist_add(hist, b)

    pltpu.sync_copy(hist_d, out_hbm.at[row, seg, 0])
    pltpu.sync_copy(hist_ds, out_hbm.at[row, seg, 1])


def _diff_hist_kernel(d_hbm, ds_hbm, p_hbm, out_hbm, hist, buf_d0, buf_d1,
                      buf_ds0, buf_ds1, pbuf, sem_d0, sem_d1, sem_ds0,
                      sem_ds1):
    c = lax.axis_index("c")
    s = lax.axis_index("s")
    row = c * (16 // NSEG) + s // NSEG
    seg = s % NSEG
    base = seg * ROWS_PER_SEG

    _zero_vmem(hist, NB)
    pltpu.sync_copy(p_hbm.at[row], pbuf)

    p = pbuf[pl.ds(0, 16)]
    lanes = lax.iota(jnp.int32, 16)

    def col(i):
        return jnp.sum(jnp.where(lanes == i, p, 0.0), axis=0)

    t_d = col(0)
    t_ds = col(1)
    inv_sd = col(2)
    inv_sds = col(3)
    scale = col(4)

    bufs_d = (buf_d0, buf_d1)
    bufs_ds = (buf_ds0, buf_ds1)
    sems_d = (sem_d0, sem_d1)
    sems_ds = (sem_ds0, sem_ds1)

    def start(ch):
        sl = pl.ds(base + ch * CROWS, CROWS)
        return (pltpu.async_copy(d_hbm.at[row, sl, :], bufs_d[ch % 2],
                                 sems_d[ch % 2]),
                pltpu.async_copy(ds_hbm.at[row, sl, :], bufs_ds[ch % 2],
                                 sems_ds[ch % 2]))

    cps = start(0)
    for ch in range(NCHUNK):
        cps[0].wait()
        cps[1].wait()
        if ch + 1 < NCHUNK:
            cps = start(ch + 1)
        bd = bufs_d[ch % 2]
        bs = bufs_ds[ch % 2]

        @plsc.parallel_loop(0, VPC, 1, unroll=8)
        def body(j):
            r = j // VPR
            col2 = (j % VPR) * 16
            xd = bd[r, pl.ds(col2, 16)]
            xs = bs[r, pl.ds(col2, 16)]
            diff = jnp.abs((xd - t_d) * inv_sd - (xs - t_ds) * inv_sds)
            b = jnp.minimum((diff * scale).astype(jnp.int32), NB - 1)
            _hist_add(hist, b)

    pltpu.sync_copy(hist, out_hbm.at[row, seg])


def _make_sc_kernels():
    mesh = _sc_mesh()
    cp = pltpu.CompilerParams(needs_layout_passes=False,
                              use_tc_tiling_on_sc=True)
    k1 = functools.partial(
        pl.kernel,
        out_type=jax.ShapeDtypeStruct((B, NSEG, 2, NB), jnp.float32),
        mesh=mesh,
        compiler_params=cp,
        scratch_types=[
            pltpu.VMEM((NB,), jnp.float32),
            pltpu.VMEM((NB,), jnp.float32),
            pltpu.VMEM((CROWS, 512), jnp.float32),
            pltpu.VMEM((CROWS, 512), jnp.float32),
            pltpu.SemaphoreType.DMA,
            pltpu.SemaphoreType.DMA,
        ],
    )(_value_hist_kernel)
    k3 = functools.partial(
        pl.kernel,
        out_type=jax.ShapeDtypeStruct((B, NSEG, NB), jnp.float32),
        mesh=mesh,
        compiler_params=cp,
        scratch_types=[
            pltpu.VMEM((NB,), jnp.float32),
            pltpu.VMEM((CROWS, 512), jnp.float32),
            pltpu.VMEM((CROWS, 512), jnp.float32),
            pltpu.VMEM((CROWS, 512), jnp.float32),
            pltpu.VMEM((CROWS, 512), jnp.float32),
            pltpu.VMEM((128,), jnp.float32),
            pltpu.SemaphoreType.DMA,
            pltpu.SemaphoreType.DMA,
            pltpu.SemaphoreType.DMA,
            pltpu.SemaphoreType.DMA,
        ],
    )(_diff_hist_kernel)
    return k1, k3


def _cum_hier(cnt8):
    nchunks = NB // 128
    nrows = B * nchunks
    cnt2 = cnt8.reshape(nrows, 128)
    i128 = lax.broadcasted_iota(jnp.int32, (128, 128), 0)
    j128 = lax.broadcasted_iota(jnp.int32, (128, 128), 1)
    tri = (i128 <= j128).astype(jnp.float32)
    lanecum = jax.lax.dot(cnt2, tri, precision=jax.lax.Precision.HIGHEST)
    ii = lax.broadcasted_iota(jnp.int32, (nrows, nrows), 0)
    jj = lax.broadcasted_iota(jnp.int32, (nrows, nrows), 1)
    prevchunk = ((ii // nchunks == jj // nchunks) & (jj < ii)).astype(jnp.float32)
    a = jax.lax.dot(prevchunk, cnt2, precision=jax.lax.Precision.HIGHEST)
    ones = jnp.ones((128, 128), jnp.float32)
    cum2 = lanecum + jax.lax.dot(a, ones, precision=jax.lax.Precision.HIGHEST)
    return cum2.reshape(B, NB)


def _rank_value(cnt, cum, bidx, k):
    below = (cum < (k + 0.5)).astype(jnp.float32)
    bstar = jnp.sum(below, axis=1, keepdims=True)
    nbelow = jnp.sum(cnt * below, axis=1, keepdims=True)
    onehot = (bidx == bstar).astype(jnp.float32)
    cntb = jnp.sum(cnt * onehot, axis=1, keepdims=True)
    cntb = jnp.maximum(cntb, 1.0)
    return bstar + (k - nbelow + 0.5) / cntb


def _params_kernel(h_ref, out_ref):
    h = h_ref[...]
    cnt_d = jnp.sum(h[:, :, 0, :], axis=1)
    cnt_ds = jnp.sum(h[:, :, 1, :], axis=1)

    bidx = lax.broadcasted_iota(jnp.int32, (B, NB), 1).astype(jnp.float32)
    width = 1.0 / NB
    centers = (bidx + 0.5) * width

    def stats(cnt):
        cum = _cum_hier(cnt)
        v0 = _rank_value(cnt, cum, bidx, float(M // 2 - 1))
        v1 = _rank_value(cnt, cum, bidx, float(M // 2))
        t = (v0 + v1) * (0.5 * width)
        absdev = jnp.sum(cnt * jnp.abs(centers - t), axis=1, keepdims=True)
        s = (jnp.sum(absdev) + 1e-08) / M
        return t, s

    t_d, s_d = stats(cnt_d)
    t_ds, s_ds = stats(cnt_ds)

    inv_sd = jnp.full((B, 1), 1.0 / s_d, jnp.float32)
    inv_sds = jnp.full((B, 1), 1.0 / s_ds, jnp.float32)
    r_bound = (jnp.maximum(t_d, 1.0 - t_d) / s_d
               + jnp.maximum(t_ds, 1.0 - t_ds) / s_ds) * (1.0 + 2.0 ** -10)
    scale = NB / r_bound
    widthd = r_bound / NB
    pad = jnp.zeros((B, 128 - 6), jnp.float32)
    out_ref[...] = jnp.concatenate(
        [t_d, t_ds, inv_sd, inv_sds, scale, widthd, pad], axis=1)


def _loss_kernel(h_ref, p_ref, out_ref):
    h = h_ref[...]
    cnt = jnp.sum(h, axis=1)
    widthd = p_ref[:, 5:6]

    cum = _cum_hier(cnt)
    bidx = lax.broadcasted_iota(jnp.int32, (B, NB), 1).astype(jnp.float32)
    below = (cum < (M80 - 0.5)).astype(jnp.float32)
    bstar = jnp.sum(below, axis=1, keepdims=True)
    nbelow = jnp.sum(cnt * below, axis=1, keepdims=True)
    onehot = (bidx == bstar).astype(jnp.float32)
    cntb = jnp.maximum(jnp.sum(cnt * onehot, axis=1, keepdims=True), 1.0)

    centers = (bidx + 0.5) * widthd
    s_below = jnp.sum(cnt * centers * below, axis=1, keepdims=True)
    m = M80 - nbelow
    left = bstar * widthd
    s_in = m * left + widthd * m * m / (2.0 * cntb)
    loss = jnp.sum(s_below + s_in) / (2.0 * M)
    out_ref[...] = loss.reshape(1, 1)


def kernel(d, d_star):
    d2 = d.reshape(B, 512, 512)
    ds2 = d_star.reshape(B, 512, 512)
    k1, k3 = _make_sc_kernels()

    h1 = k1(d2, ds2)
    params = pl.pallas_call(
        _params_kernel,
        out_shape=jax.ShapeDtypeStruct((B, 128), jnp.float32),
    )(h1)
    h2 = k3(d2, ds2, params)
    out = pl.pallas_call(
        _loss_kernel,
        out_shape=jax.ShapeDtypeStruct((1, 1), jnp.float32),
    )(h2, params)
    return out.reshape(())


# speedup vs baseline: 81.2824x; 1.0569x over previous
import functools

import jax
import jax.numpy as jnp
from jax import lax
from jax.experimental import pallas as pl
from jax.experimental.pallas import tpu as pltpu
from jax.experimental.pallas import tpu_sc as plsc

B = 8
M = 512 * 512
M80 = int(0.8 * M)
NB = 8192
NSEG = 4
ROWS_PER_SEG = 512 // NSEG
CROWS = 32
NCHUNK = ROWS_PER_SEG // CROWS
VPC = CROWS * 512 // 16
VPR = 512 // 16


def _sc_mesh():
    return plsc.VectorSubcoreMesh(core_axis_name="c", subcore_axis_name="s")


def _zero_vmem(ref, n):
    zero = jnp.zeros((16,), jnp.float32)

    def body(i, _):
        ref[pl.ds(i * 16, 16)] = zero
        return 0

    lax.fori_loop(0, n // 16, body, 0, unroll=8)


def _hist_add(hist, b):
    plsc.addupdate_scatter(hist, [b], jnp.ones((16,), jnp.float32))


def _value_hist_kernel(d_hbm, ds_hbm, out_hbm, hist_d, hist_ds, buf0, buf1,
                       sem0, sem1):
    c = lax.axis_index("c")
    s = lax.axis_index("s")
    row = c * (16 // NSEG) + s // NSEG
    seg = s % NSEG
    base = seg * ROWS_PER_SEG

    _zero_vmem(hist_d, NB)
    _zero_vmem(hist_ds, NB)

    bufs = (buf0, buf1)
    sems = (sem0, sem1)
    plan = [(d_hbm, hist_d, ch) for ch in range(NCHUNK)]
    plan += [(ds_hbm, hist_ds, ch) for ch in range(NCHUNK)]

    def start(i):
        src, _, ch = plan[i]
        return pltpu.async_copy(
            src.at[row, pl.ds(base + ch * CROWS, CROWS), :], bufs[i % 2],
            sems[i % 2])

    cp = start(0)
    for i in range(len(plan)):
        cp.wait()
        if i + 1 < len(plan):
            cp = start(i + 1)
        buf = bufs[i % 2]
        hist = plan[i][1]

        @plsc.parallel_loop(0, VPC, 1, unroll=8)
        def body(j):
            x = buf[j // VPR, pl.ds((j % VPR) * 16, 16)]
            b = jnp.minimum((x * float(NB)).astype(jnp.int32), NB - 1)
            _hist_add(hist, b)

    w = (row * NSEG + seg) * 2
    pltpu.sync_copy(hist_d, out_hbm.at[w])
    pltpu.sync_copy(hist_ds, out_hbm.at[w + 1])


def _diff_hist_kernel(d_hbm, ds_hbm, p_hbm, out_hbm, hist, buf_d0, buf_d1,
                      buf_ds0, buf_ds1, pbuf, sem_d0, sem_d1, sem_ds0,
                      sem_ds1):
    c = lax.axis_index("c")
    s = lax.axis_index("s")
    row = c * (16 // NSEG) + s // NSEG
    seg = s % NSEG
    base = seg * ROWS_PER_SEG

    _zero_vmem(hist, NB)
    pltpu.sync_copy(p_hbm.at[row], pbuf)

    p = pbuf[pl.ds(0, 16)]
    lanes = lax.iota(jnp.int32, 16)

    def col(i):
        return jnp.sum(jnp.where(lanes == i, p, 0.0), axis=0)

    t_d = col(0)
    t_ds = col(1)
    inv_sd = col(2)
    inv_sds = col(3)
    scale = col(4)

    bufs_d = (buf_d0, buf_d1)
    bufs_ds = (buf_ds0, buf_ds1)
    sems_d = (sem_d0, sem_d1)
    sems_ds = (sem_ds0, sem_ds1)

    def start(ch):
        sl = pl.ds(base + ch * CROWS, CROWS)
        return (pltpu.async_copy(d_hbm.at[row, sl, :], bufs_d[ch % 2],
                                 sems_d[ch % 2]),
                pltpu.async_copy(ds_hbm.at[row, sl, :], bufs_ds[ch % 2],
                                 sems_ds[ch % 2]))

    cps = start(0)
    for ch in range(NCHUNK):
        cps[0].wait()
        cps[1].wait()
        if ch + 1 < NCHUNK:
            cps = start(ch + 1)
        bd = bufs_d[ch % 2]
        bs = bufs_ds[ch % 2]

        @plsc.parallel_loop(0, VPC, 1, unroll=8)
        def body(j):
            r = j // VPR
            col2 = (j % VPR) * 16
            xd = bd[r, pl.ds(col2, 16)]
            xs = bs[r, pl.ds(col2, 16)]
            diff = jnp.abs((xd - t_d) * inv_sd - (xs - t_ds) * inv_sds)
            b = jnp.minimum((diff * scale).astype(jnp.int32), NB - 1)
            _hist_add(hist, b)

    pltpu.sync_copy(hist, out_hbm.at[row * NSEG + seg])


def _make_sc_kernels():
    mesh = _sc_mesh()
    cp = pltpu.CompilerParams(needs_layout_passes=False,
                              use_tc_tiling_on_sc=True)
    k1 = functools.partial(
        pl.kernel,
        out_type=jax.ShapeDtypeStruct((B * NSEG * 2, NB), jnp.float32),
        mesh=mesh,
        compiler_params=cp,
        scratch_types=[
            pltpu.VMEM((NB,), jnp.float32),
            pltpu.VMEM((NB,), jnp.float32),
            pltpu.VMEM((CROWS, 512), jnp.float32),
            pltpu.VMEM((CROWS, 512), jnp.float32),
            pltpu.SemaphoreType.DMA,
            pltpu.SemaphoreType.DMA,
        ],
    )(_value_hist_kernel)
    k3 = functools.partial(
        pl.kernel,
        out_type=jax.ShapeDtypeStruct((B * NSEG, NB), jnp.float32),
        mesh=mesh,
        compiler_params=cp,
        scratch_types=[
            pltpu.VMEM((NB,), jnp.float32),
            pltpu.VMEM((CROWS, 512), jnp.float32),
            pltpu.VMEM((CROWS, 512), jnp.float32),
            pltpu.VMEM((CROWS, 512), jnp.float32),
            pltpu.VMEM((CROWS, 512), jnp.float32),
            pltpu.VMEM((128,), jnp.float32),
            pltpu.SemaphoreType.DMA,
            pltpu.SemaphoreType.DMA,
            pltpu.SemaphoreType.DMA,
            pltpu.SemaphoreType.DMA,
        ],
    )(_diff_hist_kernel)
    return k1, k3


def _cum_hier(cnt8, tri, prevchunk, ones):
    nchunks = NB // 128
    nrows = B * nchunks
    cnt2 = cnt8.reshape(nrows, 128)
    lanecum = jax.lax.dot(cnt2, tri, precision=jax.lax.Precision.HIGHEST)
    a = jax.lax.dot(prevchunk, cnt2, precision=jax.lax.Precision.HIGHEST)
    cum2 = lanecum + jax.lax.dot(a, ones, precision=jax.lax.Precision.HIGHEST)
    return cum2.reshape(B, NB)


def _consts():
    nchunks = NB // 128
    nrows = B * nchunks
    i128 = lax.broadcasted_iota(jnp.int32, (128, 128), 0)
    j128 = lax.broadcasted_iota(jnp.int32, (128, 128), 1)
    tri = (i128 <= j128).astype(jnp.float32)
    ii = lax.broadcasted_iota(jnp.int32, (nrows, nrows), 0)
    jj = lax.broadcasted_iota(jnp.int32, (nrows, nrows), 1)
    prevchunk = ((ii // nchunks == jj // nchunks) & (jj < ii)).astype(jnp.float32)
    ones = jnp.ones((128, 128), jnp.float32)
    return tri, prevchunk, ones


def _rank_value(cnt, cum, bidx, k):
    below = (cum < (k + 0.5)).astype(jnp.float32)
    bstar = jnp.sum(below, axis=1, keepdims=True)
    nbelow = jnp.sum(cnt * below, axis=1, keepdims=True)
    onehot = (bidx == bstar).astype(jnp.float32)
    cntb = jnp.sum(cnt * onehot, axis=1, keepdims=True)
    cntb = jnp.maximum(cntb, 1.0)
    return bstar + (k - nbelow + 0.5) / cntb


def _params_kernel(h_ref, tri_ref, prev_ref, ones_ref, seld_ref,
                   selds_ref, out_ref):
    h = h_ref[...]
    tri, prevchunk, ones = tri_ref[...], prev_ref[...], ones_ref[...]
    cnt_d = jax.lax.dot(seld_ref[...], h, precision=jax.lax.Precision.HIGHEST)
    cnt_ds = jax.lax.dot(selds_ref[...], h, precision=jax.lax.Precision.HIGHEST)

    bidx = lax.broadcasted_iota(jnp.int32, (B, NB), 1).astype(jnp.float32)
    width = 1.0 / NB
    centers = (bidx + 0.5) * width

    def stats(cnt):
        cum = _cum_hier(cnt, tri, prevchunk, ones)
        v0 = _rank_value(cnt, cum, bidx, float(M // 2 - 1))
        v1 = _rank_value(cnt, cum, bidx, float(M // 2))
        t = (v0 + v1) * (0.5 * width)
        absdev = jnp.sum(cnt * jnp.abs(centers - t), axis=1, keepdims=True)
        s = (jnp.sum(absdev) + 1e-08) / M
        return t, s

    t_d, s_d = stats(cnt_d)
    t_ds, s_ds = stats(cnt_ds)

    inv_sd = jnp.full((B, 1), 1.0 / s_d, jnp.float32)
    inv_sds = jnp.full((B, 1), 1.0 / s_ds, jnp.float32)
    r_bound = (jnp.maximum(t_d, 1.0 - t_d) / s_d
               + jnp.maximum(t_ds, 1.0 - t_ds) / s_ds) * (1.0 + 2.0 ** -10)
    scale = NB / r_bound
    widthd = r_bound / NB
    pad = jnp.zeros((B, 128 - 6), jnp.float32)
    out_ref[...] = jnp.concatenate(
        [t_d, t_ds, inv_sd, inv_sds, scale, widthd, pad], axis=1)


def _loss_kernel(h_ref, p_ref, tri_ref, prev_ref, ones_ref, sel_ref,
                 out_ref):
    h = h_ref[...]
    cnt = jax.lax.dot(sel_ref[...], h, precision=jax.lax.Precision.HIGHEST)
    widthd = p_ref[:, 5:6]

    cum = _cum_hier(cnt, tri_ref[...], prev_ref[...], ones_ref[...])
    bidx = lax.broadcasted_iota(jnp.int32, (B, NB), 1).astype(jnp.float32)
    below = (cum < (M80 - 0.5)).astype(jnp.float32)
    bstar = jnp.sum(below, axis=1, keepdims=True)
    nbelow = jnp.sum(cnt * below, axis=1, keepdims=True)
    onehot = (bidx == bstar).astype(jnp.float32)
    cntb = jnp.maximum(jnp.sum(cnt * onehot, axis=1, keepdims=True), 1.0)

    centers = (bidx + 0.5) * widthd
    s_below = jnp.sum(cnt * centers * below, axis=1, keepdims=True)
    m = M80 - nbelow
    left = bstar * widthd
    s_in = m * left + widthd * m * m / (2.0 * cntb)
    loss = jnp.sum(s_below + s_in) / (2.0 * M)
    out_ref[...] = loss.reshape(1, 1)


def kernel(d, d_star):
    d2 = d.reshape(B, 512, 512)
    ds2 = d_star.reshape(B, 512, 512)
    k1, k3 = _make_sc_kernels()

    tri, prevchunk, ones = _consts()
    iw = lax.broadcasted_iota(jnp.int32, (B, B * NSEG * 2), 0)
    jw = lax.broadcasted_iota(jnp.int32, (B, B * NSEG * 2), 1)
    sel_d = ((jw // (NSEG * 2) == iw) & (jw % 2 == 0)).astype(jnp.float32)
    sel_ds = ((jw // (NSEG * 2) == iw) & (jw % 2 == 1)).astype(jnp.float32)
    i2 = lax.broadcasted_iota(jnp.int32, (B, B * NSEG), 0)
    j2 = lax.broadcasted_iota(jnp.int32, (B, B * NSEG), 1)
    sel2 = (j2 // NSEG == i2).astype(jnp.float32)

    h1 = k1(d2, ds2)
    params = pl.pallas_call(
        _params_kernel,
        out_shape=jax.ShapeDtypeStruct((B, 128), jnp.float32),
    )(h1, tri, prevchunk, ones, sel_d, sel_ds)
    h2 = k3(d2, ds2, params)
    out = pl.pallas_call(
        _loss_kernel,
        out_shape=jax.ShapeDtypeStruct((1, 1), jnp.float32),
    )(h2, params, tri, prevchunk, ones, sel2)
    return out.reshape(())
